# baseline (speedup 1.0000x reference)
import functools

import jax
import jax.numpy as jnp
import numpy as np
from jax.experimental import pallas as pl
from jax.experimental.pallas import tpu as pltpu

_WIDTH = 64
_K = 8192
_TEMP = 0.5
_TN = 224


def _dot(a, b):
    return jax.lax.dot_general(a, b, (((1,), (0,)), ((), ())),
                               preferred_element_type=jnp.float32)


def _quant_block(scale_ref, z_ref, caug_ref, csqr_ref, zq_ref, pt_ref, lse_ref):
    @pl.when(pl.program_id(0) == 0)
    def _init():
        pt_ref[...] = jnp.zeros((1, 1), jnp.float32)
        lse_ref[...] = jnp.zeros((1, 1), jnp.float32)

    z = z_ref[...]
    caug = caug_ref[...]
    csqr = csqr_ref[...]
    scale = scale_ref[0]

    zsq = jnp.sum(z * z, axis=1, keepdims=True)
    s = jax.lax.dot_general(z, caug[:, :_WIDTH], (((1,), (1,)), ((), ())),
                            preferred_element_type=jnp.float32)
    g = csqr - 2.0 * s
    mg = jnp.min(g, axis=1, keepdims=True)
    e = jnp.exp((g - mg) * scale)
    r = _dot(e, caug)
    den = r[:, _WIDTH + 1:_WIDTH + 2]
    ecsq = r[:, _WIDTH:_WIDTH + 1]
    zq = r[:, :_WIDTH] / den
    zq_ref[...] = zq
    zdotzq = jnp.sum(z * zq, axis=1, keepdims=True)
    pt_row = scale * (zsq + ecsq / den - 2.0 * zdotzq)
    m = scale * (zsq + mg)
    lse_row = jnp.log(den) + m
    pt_ref[...] += jnp.sum(pt_row).reshape(1, 1)
    lse_ref[...] += jnp.sum(lse_row).reshape(1, 1)


def _quantize(zf, codebook, var):
    n = zf.shape[0]
    k = codebook.shape[0]
    scale = (-0.5 / (var * _TEMP)).reshape(1).astype(jnp.float32)
    csq = jnp.sum(codebook * codebook, axis=1)
    caug = jnp.concatenate(
        [codebook, csq[:, None], jnp.ones((k, 1), jnp.float32),
         jnp.zeros((k, 128 - _WIDTH - 2), jnp.float32)], axis=1)
    csqr = csq[None, :]
    zq, pt, lse = pl.pallas_call(
        _quant_block,
        grid=(n // _TN,),
        in_specs=[
            pl.BlockSpec(memory_space=pltpu.SMEM),
            pl.BlockSpec((_TN, _WIDTH), lambda i: (i, 0)),
            pl.BlockSpec((_K, 128), lambda i: (0, 0)),
            pl.BlockSpec((1, _K), lambda i: (0, 0)),
        ],
        out_specs=[
            pl.BlockSpec((_TN, _WIDTH), lambda i: (i, 0)),
            pl.BlockSpec((1, 1), lambda i: (0, 0)),
            pl.BlockSpec((1, 1), lambda i: (0, 0)),
        ],
        out_shape=[
            jax.ShapeDtypeStruct((n, _WIDTH), jnp.float32),
            jax.ShapeDtypeStruct((1, 1), jnp.float32),
            jax.ShapeDtypeStruct((1, 1), jnp.float32),
        ],
    )(scale, zf, caug, csqr)
    return zq, pt[0, 0], lse[0, 0]


def _mm_bias_relu_kernel(p_ref, w_ref, b_ref, o_ref):
    o_ref[0] = jnp.maximum(_dot(p_ref[0], w_ref[...]) + b_ref[...], 0.0)


def _enc_conv_kernel(pl_ref, w_ref, b_ref, o_ref, *, oh, wp, relu):
    acc = None
    for sy in (0, 1):
        for sx in (0, 1):
            plane = pl_ref[0, sy, sx]
            for dy in (0, 1):
                for dx in (0, 1):
                    t = (2 * dy + sy) * 4 + (2 * dx + sx)
                    sl = plane[dy * wp + dx:dy * wp + dx + oh * wp]
                    part = _dot(sl, w_ref[t])
                    acc = part if acc is None else acc + part
    y = acc + b_ref[...]
    o_ref[0] = jnp.maximum(y, 0.0) if relu else y


def _enc_conv(planes, w16, b, oh, wp, relu):
    bsz = planes.shape[0]
    c = planes.shape[-1]
    o = w16.shape[-1]
    m = oh * wp
    return pl.pallas_call(
        functools.partial(_enc_conv_kernel, oh=oh, wp=wp, relu=relu),
        grid=(bsz,),
        in_specs=[
            pl.BlockSpec((1, 2, 2, (oh + 1) * wp + 8, c),
                         lambda i: (i, 0, 0, 0, 0)),
            pl.BlockSpec((16, c, o), lambda i: (0, 0, 0)),
            pl.BlockSpec((1, o), lambda i: (0, 0)),
        ],
        out_specs=pl.BlockSpec((1, m, o), lambda i: (i, 0, 0)),
        out_shape=jax.ShapeDtypeStruct((bsz, m, o), jnp.float32),
    )(planes, w16, b[None, :])


def _tconv_kernel(xp_ref, w_ref, b_ref, o_ref, *, h, wpp, relu):
    acc = None
    xp = xp_ref[0]
    for u in (0, 1, 2):
        for v in (0, 1, 2):
            sl = xp[u * wpp + v:u * wpp + v + h * wpp]
            part = _dot(sl, w_ref[3 * u + v])
            acc = part if acc is None else acc + part
    y = acc + b_ref[...]
    o_ref[0] = jnp.maximum(y, 0.0) if relu else y


def _tconv(xpf, wc, b4, h, wpp, relu):
    bsz = xpf.shape[0]
    c = xpf.shape[-1]
    o4 = wc.shape[-1]
    m = h * wpp
    return pl.pallas_call(
        functools.partial(_tconv_kernel, h=h, wpp=wpp, relu=relu),
        grid=(bsz,),
        in_specs=[
            pl.BlockSpec((1, (h + 2) * wpp + 8, c), lambda i: (i, 0, 0)),
            pl.BlockSpec((9, c, o4), lambda i: (0, 0, 0)),
            pl.BlockSpec((1, o4), lambda i: (0, 0)),
        ],
        out_specs=pl.BlockSpec((1, m, o4), lambda i: (i, 0, 0)),
        out_shape=jax.ShapeDtypeStruct((bsz, m, o4), jnp.float32),
    )(xpf, wc, b4[None, :])


def _to_planes(act):
    bsz, h, w, c = act.shape
    p = jnp.pad(act, ((0, 0), (1, 1), (1, 1), (0, 0)))
    p = p.reshape(bsz, (h + 2) // 2, 2, (w + 2) // 2, 2, c)
    p = p.transpose(0, 2, 4, 1, 3, 5)
    p = p.reshape(bsz, 2, 2, ((h + 2) // 2) * ((w + 2) // 2), c)
    return jnp.pad(p, ((0, 0), (0, 0), (0, 0), (0, 8), (0, 0)))


def _frame_to_img(frame, oh, ow):
    bsz, _, o = frame.shape
    return frame.reshape(bsz, oh, ow + 1, o)[:, :, :ow]


def _interleave(frame, h, w, o):
    bsz = frame.shape[0]
    y = frame.reshape(bsz, h, w + 2, 2, 2, o)[:, :, :w]
    y = y.transpose(0, 1, 3, 2, 4, 5)
    return y.reshape(bsz, 2 * h, 2 * w, o)


def _enc_w16(w):
    return w.transpose(2, 3, 1, 0).reshape(16, w.shape[1], w.shape[0])


def _tconv_w9(w):
    o, i = w.shape[0], w.shape[1]
    wc = jnp.zeros((9, i, 4 * o), jnp.float32)
    for a in (0, 1):
        for b in (0, 1):
            for dy in (0, 1):
                for dx in (0, 1):
                    u, v = a + dy, b + dx
                    ky, kx = 2 * dy + a, 2 * dx + b
                    blk = w[:, :, 3 - ky, 3 - kx].T
                    j = (a * 2 + b) * o
                    wc = wc.at[3 * u + v, :, j:j + o].add(blk)
    return wc


def kernel(x, enc_w1, enc_b1, enc_w2, enc_b2, enc_w3, enc_b3,
           dec_w1, dec_b1, dec_w2, dec_b2, dec_w3, dec_b3, codebook, log_var):
    bsz = x.shape[0]
    p1 = jax.lax.conv_general_dilated_patches(
        x, (4, 4), (2, 2), ((1, 1), (1, 1)),
        dimension_numbers=('NCHW', 'OIHW', 'NCHW'))
    p1 = p1.transpose(0, 2, 3, 1).reshape(bsz, 112 * 112, 48)
    wm1 = enc_w1.transpose(1, 2, 3, 0).reshape(48, _WIDTH)
    h1 = pl.pallas_call(
        _mm_bias_relu_kernel,
        grid=(bsz,),
        in_specs=[
            pl.BlockSpec((1, 112 * 112, 48), lambda i: (i, 0, 0)),
            pl.BlockSpec((48, _WIDTH), lambda i: (0, 0)),
            pl.BlockSpec((1, _WIDTH), lambda i: (0, 0)),
        ],
        out_specs=pl.BlockSpec((1, 112 * 112, _WIDTH), lambda i: (i, 0, 0)),
        out_shape=jax.ShapeDtypeStruct((bsz, 112 * 112, _WIDTH), jnp.float32),
    )(p1, wm1, enc_b1[None, :])
    h1 = h1.reshape(bsz, 112, 112, _WIDTH)

    h2 = _enc_conv(_to_planes(h1), _enc_w16(enc_w2), enc_b2, 56, 57, True)
    h2 = _frame_to_img(h2, 56, 56)
    z = _enc_conv(_to_planes(h2), _enc_w16(enc_w3), enc_b3, 28, 29, False)
    z = _frame_to_img(z, 28, 28)
    zf = z.reshape(bsz * 28 * 28, _WIDTH)

    var = jnp.exp(log_var)
    zq, pt_sum, lse_sum = _quantize(zf, codebook, var)
    n = zf.shape[0]
    mean_pt = pt_sum / n
    mean_lse = lse_sum / n
    loss_latent = (1.0 - _TEMP) * mean_pt - mean_lse + np.float32(np.log(_K))

    def xpad(a):
        bb, hh, ww, cc = a.shape
        f = jnp.pad(a, ((0, 0), (1, 1), (1, 1), (0, 0))).reshape(
            bb, (hh + 2) * (ww + 2), cc)
        return jnp.pad(f, ((0, 0), (0, 8), (0, 0)))

    zq4 = zq.reshape(bsz, 28, 28, _WIDTH)
    d1 = _tconv(xpad(zq4), _tconv_w9(dec_w1), jnp.tile(dec_b1, 4), 28, 30, True)
    a1 = _interleave(d1, 28, 28, _WIDTH)
    d2 = _tconv(xpad(a1), _tconv_w9(dec_w2), jnp.tile(dec_b2, 4), 56, 58, True)
    a2 = _interleave(d2, 56, 56, _WIDTH)
    d3 = _tconv(xpad(a2), _tconv_w9(dec_w3), jnp.tile(dec_b3, 4), 112, 114,
                False)
    xr = _interleave(d3, 112, 112, 3)
    x_rec = xr.transpose(0, 3, 1, 2)

    dim_x = float(np.prod(x_rec.shape[1:]))
    se = jnp.sum((x_rec - x) ** 2) / bsz
    loss_rec = dim_x * jnp.log(se) / 2.0
    rmse = jnp.sqrt(se / dim_x)
    loss = loss_latent + loss_rec
    return (loss, x_rec, rmse)


# speedup vs baseline: 2.2819x; 2.2819x over previous
import jax
import jax.numpy as jnp
import numpy as np
from jax.experimental import pallas as pl
from jax.experimental.pallas import tpu as pltpu

_WIDTH = 64
_K = 8192
_TEMP = 0.5
_TN = 224


def _dot(a, b):
    return jax.lax.dot_general(a, b, (((1,), (0,)), ((), ())),
                               preferred_element_type=jnp.float32)


def _quant_block(scale_ref, z_ref, caug_ref, csqr_ref, zq_ref, pt_ref, lse_ref):
    @pl.when(pl.program_id(0) == 0)
    def _init():
        pt_ref[...] = jnp.zeros((1, 1), jnp.float32)
        lse_ref[...] = jnp.zeros((1, 1), jnp.float32)

    z = z_ref[...]
    caug = caug_ref[...]
    csqr = csqr_ref[...]
    scale = scale_ref[0]

    zsq = jnp.sum(z * z, axis=1, keepdims=True)
    s = jax.lax.dot_general(z, caug[:, :_WIDTH], (((1,), (1,)), ((), ())),
                            preferred_element_type=jnp.float32)
    g = csqr - 2.0 * s
    mg = jnp.min(g, axis=1, keepdims=True)
    e = jnp.exp((g - mg) * scale)
    r = _dot(e, caug)
    den = r[:, _WIDTH + 1:_WIDTH + 2]
    ecsq = r[:, _WIDTH:_WIDTH + 1]
    zq = r[:, :_WIDTH] / den
    zq_ref[...] = zq
    zdotzq = jnp.sum(z * zq, axis=1, keepdims=True)
    pt_row = scale * (zsq + ecsq / den - 2.0 * zdotzq)
    m = scale * (zsq + mg)
    lse_row = jnp.log(den) + m
    pt_ref[...] += jnp.sum(pt_row).reshape(1, 1)
    lse_ref[...] += jnp.sum(lse_row).reshape(1, 1)


def _quantize(zf, codebook, var):
    n = zf.shape[0]
    k = codebook.shape[0]
    scale = (-0.5 / (var * _TEMP)).reshape(1).astype(jnp.float32)
    csq = jnp.sum(codebook * codebook, axis=1)
    caug = jnp.concatenate(
        [codebook, csq[:, None], jnp.ones((k, 1), jnp.float32),
         jnp.zeros((k, 128 - _WIDTH - 2), jnp.float32)], axis=1)
    csqr = csq[None, :]
    zq, pt, lse = pl.pallas_call(
        _quant_block,
        grid=(n // _TN,),
        in_specs=[
            pl.BlockSpec(memory_space=pltpu.SMEM),
            pl.BlockSpec((_TN, _WIDTH), lambda i: (i, 0)),
            pl.BlockSpec((_K, 128), lambda i: (0, 0)),
            pl.BlockSpec((1, _K), lambda i: (0, 0)),
        ],
        out_specs=[
            pl.BlockSpec((_TN, _WIDTH), lambda i: (i, 0)),
            pl.BlockSpec((1, 1), lambda i: (0, 0)),
            pl.BlockSpec((1, 1), lambda i: (0, 0)),
        ],
        out_shape=[
            jax.ShapeDtypeStruct((n, _WIDTH), jnp.float32),
            jax.ShapeDtypeStruct((1, 1), jnp.float32),
            jax.ShapeDtypeStruct((1, 1), jnp.float32),
        ],
    )(scale, zf, caug, csqr)
    return zq, pt[0, 0], lse[0, 0]


_NHWC = ('NHWC', 'HWIO', 'NHWC')


def _conv_s2(x, w, b):
    y = jax.lax.conv_general_dilated(x, w.transpose(2, 3, 1, 0), (2, 2),
                                     ((1, 1), (1, 1)), dimension_numbers=_NHWC)
    return y + b[None, None, None, :]


def _tconv_s2(x, w, b):
    wf = w[:, :, ::-1, ::-1].transpose(2, 3, 1, 0)
    y = jax.lax.conv_general_dilated(x, wf, (1, 1), ((2, 2), (2, 2)),
                                     lhs_dilation=(2, 2),
                                     dimension_numbers=_NHWC)
    return y + b[None, None, None, :]


def kernel(x, enc_w1, enc_b1, enc_w2, enc_b2, enc_w3, enc_b3,
           dec_w1, dec_b1, dec_w2, dec_b2, dec_w3, dec_b3, codebook, log_var):
    bsz = x.shape[0]
    xh = x.transpose(0, 2, 3, 1)
    h = jax.nn.relu(_conv_s2(xh, enc_w1, enc_b1))
    h = jax.nn.relu(_conv_s2(h, enc_w2, enc_b2))
    z = _conv_s2(h, enc_w3, enc_b3)
    zf = z.reshape(bsz * 28 * 28, _WIDTH)

    var = jnp.exp(log_var)
    zq, pt_sum, lse_sum = _quantize(zf, codebook, var)
    n = zf.shape[0]
    mean_pt = pt_sum / n
    mean_lse = lse_sum / n
    loss_latent = (1.0 - _TEMP) * mean_pt - mean_lse + np.float32(np.log(_K))

    zq4 = zq.reshape(bsz, 28, 28, _WIDTH)
    h = jax.nn.relu(_tconv_s2(zq4, dec_w1, dec_b1))
    h = jax.nn.relu(_tconv_s2(h, dec_w2, dec_b2))
    xr = _tconv_s2(h, dec_w3, dec_b3)
    x_rec = xr.transpose(0, 3, 1, 2)

    dim_x = float(np.prod(x_rec.shape[1:]))
    se = jnp.sum((x_rec - x) ** 2) / bsz
    loss_rec = dim_x * jnp.log(se) / 2.0
    rmse = jnp.sqrt(se / dim_x)
    loss = loss_latent + loss_rec
    return (loss, x_rec, rmse)
